# baseline (speedup 1.0000x reference)
import functools

import jax
import jax.numpy as jnp
from jax import lax
from jax.experimental import pallas as pl
from jax.experimental.pallas import tpu as pltpu
from jax.experimental.pallas import tpu_sc as plsc


def _make_row_gather(B, N, F):
    info = plsc.get_sparse_core_info()
    nc, ns, L = info.num_cores, info.num_subcores, info.num_lanes
    nw = nc * ns
    assert B % (8 * nw) == 0
    bpw = B // nw
    PHYS = ((N + 7) // 8) * 8
    WPAD = ((N + L - 1) // L) * L
    nchunk = WPAD // L

    mesh = plsc.VectorSubcoreMesh(core_axis_name="c", subcore_axis_name="s")

    def body(adj2, no2, aid, adj_row, no_row,
             aid_v, ra_v, rb_v, s_v, idx_v, rows2_v, nrows_v, out_v, sem, sem2):
        wid = lax.axis_index("s") * nc + lax.axis_index("c")
        base = wid * bpw
        pltpu.sync_copy(aid.at[pl.ds(base, bpw)], aid_v)
        for j in range(bpw // L):
            ids = aid_v[pl.ds(j * L, L)]
            r_star = ((base + j * L) + lax.iota(jnp.int32, L)) * N + ids
            w = r_star * PHYS
            ra = lax.div(w, jnp.int32(N))
            s = w - ra * N
            ra_v[pl.ds(j * L, L)] = ra
            rb_v[pl.ds(j * L, L)] = ra + 1
            s_v[pl.ds(j * L, L)] = s
            idx_v[pl.ds(j * L, L)] = r_star
        cpa = pltpu.async_copy(adj2.at[ra_v], rows2_v.at[pl.ds(0, bpw)], sem)
        cpb = pltpu.async_copy(adj2.at[rb_v], rows2_v.at[pl.ds(bpw, bpw)], sem)
        cpn = pltpu.async_copy(no2.at[idx_v], nrows_v, sem2)
        cpa.wait()
        cpb.wait()

        def realign(i, carry):
            i_vec = jnp.zeros((L,), jnp.int32) + i
            s16 = plsc.load_gather(s_v, [i_vec])
            for c in range(nchunk):
                p = s16 + (c * L + lax.iota(jnp.int32, L))
                zero = jnp.zeros((L,), jnp.int32)
                ge1 = jnp.where(p >= N, zero + 1, zero)
                ge2 = jnp.where(p >= 2 * N, zero + 1, zero)
                col = p - N * (ge1 + ge2)
                t = N * i_vec + col + ge1 * (PHYS * bpw)
                row2 = lax.div(t, jnp.int32(PHYS))
                col2 = t - row2 * PHYS
                v = plsc.load_gather(rows2_v, [row2, col2])
                out_v[pl.ds(i * WPAD + c * L, L)] = v
            return carry

        lax.fori_loop(0, bpw, realign, 0)
        pltpu.sync_copy(out_v, adj_row.at[pl.ds(base * WPAD, bpw * WPAD)])
        cpn.wait()
        pltpu.sync_copy(nrows_v, no_row.at[pl.ds(base, bpw)])

    return pl.kernel(
        body,
        out_type=(
            jax.ShapeDtypeStruct((B * WPAD,), jnp.float32),
            jax.ShapeDtypeStruct((B, F), jnp.float32),
        ),
        mesh=mesh,
        scratch_types=[
            pltpu.VMEM((bpw,), jnp.int32),
            pltpu.VMEM((bpw,), jnp.int32),
            pltpu.VMEM((bpw,), jnp.int32),
            pltpu.VMEM((bpw,), jnp.int32),
            pltpu.VMEM((bpw,), jnp.int32),
            pltpu.VMEM((2 * bpw, N), jnp.float32),
            pltpu.VMEM((bpw, F), jnp.float32),
            pltpu.VMEM((bpw * WPAD,), jnp.float32),
            pltpu.SemaphoreType.DMA,
            pltpu.SemaphoreType.DMA,
        ],
        compiler_params=pltpu.CompilerParams(
            use_tc_tiling_on_sc=False, needs_layout_passes=False),
        name="sc_row_gather",
    )


def _dense_body(N, w_ref, nrow_ref, obs_ref, nodes_ref, Wi_ref, bi_ref,
                Wm_ref, Wu_ref, W1_ref, b1_ref, W2_ref, b2_ref, Wq_ref,
                bq_ref, q_ref):
    Bb = w_ref.shape[0]
    HG = Wi_ref.shape[1]
    x = nodes_ref[...]
    Wi = Wi_ref[...]
    bi = bi_ref[...]
    h = jnp.maximum(jnp.dot(x, Wi, preferred_element_type=jnp.float32) + bi, 0.0)
    msg = jnp.maximum(jnp.dot(h, Wm_ref[...], preferred_element_type=jnp.float32), 0.0)
    msg3 = msg.reshape(Bb, N, HG)
    w = w_ref[:, :N]
    w = jnp.where(w > 0.0, w, 0.0)
    agg = jnp.sum(msg3 * w[:, :, None], axis=1)
    hrow = jnp.maximum(
        jnp.dot(nrow_ref[...], Wi, preferred_element_type=jnp.float32) + bi, 0.0)
    h2 = jnp.maximum(
        hrow + jnp.dot(agg, Wu_ref[...], preferred_element_type=jnp.float32), 0.0)
    inp = jnp.concatenate([obs_ref[...], h2], axis=1)
    z = jnp.maximum(
        jnp.dot(inp, W1_ref[...], preferred_element_type=jnp.float32) + b1_ref[...], 0.0)
    z = jnp.maximum(
        jnp.dot(z, W2_ref[...], preferred_element_type=jnp.float32) + b2_ref[...], 0.0)
    q_ref[...] = jnp.dot(z, Wq_ref[...], preferred_element_type=jnp.float32) + bq_ref[...]


def kernel(obs, rnn_states, node_obs, adj, agent_id, W_in, b_in, W_msg,
           W_upd, W1, b1, W2, b2, Wq, bq):
    B, N, F = node_obs.shape
    OBS = obs.shape[1]
    HG = W_in.shape[1]
    HID = W1.shape[1]
    ACT = Wq.shape[1]
    WPAD = 112

    aid = agent_id.reshape(B).astype(jnp.int32)
    adj2 = adj.reshape(B * N, N)
    no2 = node_obs.reshape(B * N, F)
    adj_row_flat, no_row = _make_row_gather(B, N, F)(adj2, no2, aid)
    adj_row = adj_row_flat.reshape(B, WPAD)

    Bb = 128
    grid = (B // Bb,)
    full = lambda shape: pl.BlockSpec(shape, lambda i: (0,) * len(shape))
    q = pl.pallas_call(
        functools.partial(_dense_body, N),
        grid=grid,
        in_specs=[
            pl.BlockSpec((Bb, WPAD), lambda i: (i, 0)),
            pl.BlockSpec((Bb, F), lambda i: (i, 0)),
            pl.BlockSpec((Bb, OBS), lambda i: (i, 0)),
            pl.BlockSpec((Bb * N, F), lambda i: (i, 0)),
            full((F, HG)),
            full((1, HG)),
            full((HG, HG)),
            full((HG, HG)),
            full((OBS + HG, HID)),
            full((1, HID)),
            full((HID, HID)),
            full((1, HID)),
            full((HID, ACT)),
            full((1, ACT)),
        ],
        out_specs=pl.BlockSpec((Bb, ACT), lambda i: (i, 0)),
        out_shape=jax.ShapeDtypeStruct((B, ACT), jnp.float32),
    )(adj_row, no_row, obs, no2, W_in, b_in.reshape(1, HG), W_msg, W_upd,
      W1, b1.reshape(1, HID), W2, b2.reshape(1, HID), Wq, bq.reshape(1, ACT))

    return (q, rnn_states)


# speedup vs baseline: 1.6343x; 1.6343x over previous
import functools

import jax
import jax.numpy as jnp
from jax import lax
from jax.experimental import pallas as pl
from jax.experimental.pallas import tpu as pltpu
from jax.experimental.pallas import tpu_sc as plsc


def _make_node_gather(B, N, F):
    info = plsc.get_sparse_core_info()
    nc, ns, L = info.num_cores, info.num_subcores, info.num_lanes
    nw = nc * ns
    assert B % (8 * nw) == 0
    bpw = B // nw
    mesh = plsc.VectorSubcoreMesh(core_axis_name="c", subcore_axis_name="s")

    def body(no2, aid, no_row, aid_v, idx_v, nrows_v, sem):
        wid = lax.axis_index("s") * nc + lax.axis_index("c")
        base = wid * bpw
        pltpu.sync_copy(aid.at[pl.ds(base, bpw)], aid_v)
        for j in range(bpw // L):
            ids = aid_v[pl.ds(j * L, L)]
            r_star = ((base + j * L) + lax.iota(jnp.int32, L)) * N + ids
            idx_v[pl.ds(j * L, L)] = r_star
        pltpu.async_copy(no2.at[idx_v], nrows_v, sem).wait()
        pltpu.sync_copy(nrows_v, no_row.at[pl.ds(base, bpw)])

    return pl.kernel(
        body,
        out_type=jax.ShapeDtypeStruct((B, F), jnp.float32),
        mesh=mesh,
        scratch_types=[
            pltpu.VMEM((bpw,), jnp.int32),
            pltpu.VMEM((bpw,), jnp.int32),
            pltpu.VMEM((bpw, F), jnp.float32),
            pltpu.SemaphoreType.DMA,
        ],
        compiler_params=pltpu.CompilerParams(
            use_tc_tiling_on_sc=False, needs_layout_passes=False),
        name="sc_node_row_gather",
    )


def _dense_body(N, adj_ref, aid_ref, nrow_ref, obs_ref, nodes_ref, Wi_ref,
                bi_ref, Wm_ref, Wu_ref, W1_ref, b1_ref, W2_ref, b2_ref,
                Wq_ref, bq_ref, q_ref):
    Bb = aid_ref.shape[0]
    HG = Wi_ref.shape[1]
    x = nodes_ref[...]
    Wi = Wi_ref[...]
    bi = bi_ref[...]
    h = jnp.maximum(jnp.dot(x, Wi, preferred_element_type=jnp.float32) + bi, 0.0)
    msg = jnp.maximum(jnp.dot(h, Wm_ref[...], preferred_element_type=jnp.float32), 0.0)
    msg3 = msg.reshape(Bb, N, HG)
    aid_i = aid_ref[...]
    node_iota = lax.broadcasted_iota(jnp.int32, (Bb, N), 1)
    onehot = jnp.where(node_iota == aid_i, 1.0, 0.0)
    w = jnp.sum(adj_ref[...] * onehot[:, :, None], axis=1)
    w = jnp.where(w > 0.0, w, 0.0)
    agg = jnp.sum(msg3 * w[:, :, None], axis=1)
    hrow = jnp.maximum(
        jnp.dot(nrow_ref[...], Wi, preferred_element_type=jnp.float32) + bi, 0.0)
    h2 = jnp.maximum(
        hrow + jnp.dot(agg, Wu_ref[...], preferred_element_type=jnp.float32), 0.0)
    inp = jnp.concatenate([obs_ref[...], h2], axis=1)
    z = jnp.maximum(
        jnp.dot(inp, W1_ref[...], preferred_element_type=jnp.float32) + b1_ref[...], 0.0)
    z = jnp.maximum(
        jnp.dot(z, W2_ref[...], preferred_element_type=jnp.float32) + b2_ref[...], 0.0)
    q_ref[...] = jnp.dot(z, Wq_ref[...], preferred_element_type=jnp.float32) + bq_ref[...]


def kernel(obs, rnn_states, node_obs, adj, agent_id, W_in, b_in, W_msg,
           W_upd, W1, b1, W2, b2, Wq, bq):
    B, N, F = node_obs.shape
    OBS = obs.shape[1]
    HG = W_in.shape[1]
    HID = W1.shape[1]
    ACT = Wq.shape[1]

    aid = agent_id.reshape(B).astype(jnp.int32)
    no2 = node_obs.reshape(B * N, F)
    no_row = _make_node_gather(B, N, F)(no2, aid)
    aid_f = aid.reshape(B, 1)

    Bb = 128
    grid = (B // Bb,)
    full = lambda shape: pl.BlockSpec(shape, lambda i: (0,) * len(shape))
    q = pl.pallas_call(
        functools.partial(_dense_body, N),
        grid=grid,
        in_specs=[
            pl.BlockSpec((Bb, N, N), lambda i: (i, 0, 0)),
            pl.BlockSpec((Bb, 1), lambda i: (i, 0)),
            pl.BlockSpec((Bb, F), lambda i: (i, 0)),
            pl.BlockSpec((Bb, OBS), lambda i: (i, 0)),
            pl.BlockSpec((Bb * N, F), lambda i: (i, 0)),
            full((F, HG)),
            full((1, HG)),
            full((HG, HG)),
            full((HG, HG)),
            full((OBS + HG, HID)),
            full((1, HID)),
            full((HID, HID)),
            full((1, HID)),
            full((HID, ACT)),
            full((1, ACT)),
        ],
        out_specs=pl.BlockSpec((Bb, ACT), lambda i: (i, 0)),
        out_shape=jax.ShapeDtypeStruct((B, ACT), jnp.float32),
    )(adj, aid_f, no_row, obs, no2, W_in, b_in.reshape(1, HG), W_msg, W_upd,
      W1, b1.reshape(1, HID), W2, b2.reshape(1, HID), Wq, bq.reshape(1, ACT))

    return (q, rnn_states)
